# Initial kernel scaffold; baseline (speedup 1.0000x reference)
import functools

import jax
import jax.numpy as jnp
from jax import lax
from jax.experimental import pallas as pl
from jax.experimental.pallas import tpu as pltpu
from jax.experimental.pallas import tpu_sc as plsc

_HID = 64
_B = 4096
_S = 200
_HALF = _S // 2
_NC = 2
_NS = 16
_NW = _NC * _NS
_ROWS_PER_W = _B // _NW


def _body(ids_hbm, tok_hbm, pos_hbm, out_hbm, pos_v, idx_v, rows_v, sem):
    w = lax.axis_index("s") * _NC + lax.axis_index("c")
    base = w * _ROWS_PER_W

    pltpu.sync_copy(pos_hbm, pos_v)

    @pl.loop(0, _ROWS_PER_W)
    def _(i):
        r = base + i
        pltpu.sync_copy(ids_hbm.at[r], idx_v)
        pltpu.sync_copy(pos_v, rows_v)
        d0 = pltpu.async_copy(
            tok_hbm.at[idx_v.at[0]], rows_v.at[pl.ds(0, _HALF)], sem, add=True
        )
        d1 = pltpu.async_copy(
            tok_hbm.at[idx_v.at[1]], rows_v.at[pl.ds(_HALF, _HALF)], sem, add=True
        )
        d0.wait()
        d1.wait()
        pltpu.sync_copy(rows_v, out_hbm.at[r])


@jax.jit
def _run(ids3, token_embedding, pos_s):
    mesh = plsc.VectorSubcoreMesh(
        core_axis_name="c", subcore_axis_name="s", num_cores=_NC, num_subcores=_NS
    )
    return pl.kernel(
        _body,
        out_type=jax.ShapeDtypeStruct((_B, _S, _HID), jnp.float32),
        mesh=mesh,
        scratch_types=[
            pltpu.VMEM((_S, _HID), jnp.float32),
            pltpu.VMEM((2, _HALF), jnp.int32),
            pltpu.VMEM((_S, _HID), jnp.float32),
            pltpu.SemaphoreType.DMA,
        ],
    )(ids3, token_embedding, pos_s)


def kernel(input_ids, token_embedding, position_embedding):
    ids3 = input_ids.reshape(_B, 2, _HALF)
    pos_s = position_embedding[:_S]
    return _run(ids3, token_embedding, pos_s)


# speedup vs baseline: 3.1754x; 3.1754x over previous
import functools

import jax
import jax.numpy as jnp
from jax import lax
from jax.experimental import pallas as pl
from jax.experimental.pallas import tpu as pltpu
from jax.experimental.pallas import tpu_sc as plsc

_HID = 64
_B = 4096
_S = 200
_HALF = _S // 2
_NC = 2
_NS = 16
_NW = _NC * _NS
_ROWS_PER_W = _B // _NW


def _body(ids_hbm, tok_hbm, pos_hbm, out_hbm, pos_sh, idx_v, rows_v, sem):
    sid = lax.axis_index("s")
    w = sid * _NC + lax.axis_index("c")
    base = w * _ROWS_PER_W

    @pl.when(sid == 0)
    def _():
        pltpu.sync_copy(pos_hbm, pos_sh)

    plsc.subcore_barrier()

    @pl.loop(0, _ROWS_PER_W)
    def _(i):
        r = base + i
        pltpu.sync_copy(ids_hbm.at[r], idx_v)
        pltpu.sync_copy(pos_sh, rows_v)
        d0 = pltpu.async_copy(
            tok_hbm.at[idx_v.at[0]], rows_v.at[pl.ds(0, _HALF)], sem, add=True
        )
        d1 = pltpu.async_copy(
            tok_hbm.at[idx_v.at[1]], rows_v.at[pl.ds(_HALF, _HALF)], sem, add=True
        )
        d0.wait()
        d1.wait()
        pltpu.sync_copy(rows_v, out_hbm.at[r])


@jax.jit
def _run(ids3, token_embedding, pos_s):
    mesh = plsc.VectorSubcoreMesh(
        core_axis_name="c", subcore_axis_name="s", num_cores=_NC, num_subcores=_NS
    )
    return pl.kernel(
        _body,
        out_type=jax.ShapeDtypeStruct((_B, _S, _HID), jnp.float32),
        mesh=mesh,
        compiler_params=pltpu.CompilerParams(use_tc_tiling_on_sc=False),
        scratch_types=[
            pltpu.VMEM_SHARED((_S, _HID), jnp.float32),
            pltpu.VMEM((2, _HALF), jnp.int32),
            pltpu.VMEM((_S, _HID), jnp.float32),
            pltpu.SemaphoreType.DMA,
        ],
    )(ids3, token_embedding, pos_s)


def kernel(input_ids, token_embedding, position_embedding):
    ids3 = input_ids.reshape(_B, 2, _HALF)
    pos_s = position_embedding[:_S]
    return _run(ids3, token_embedding, pos_s)


# speedup vs baseline: 3.9061x; 1.2301x over previous
import jax
import jax.numpy as jnp
from jax import lax
from jax.experimental import pallas as pl
from jax.experimental.pallas import tpu as pltpu
from jax.experimental.pallas import tpu_sc as plsc

_HID = 64
_B = 4096
_S = 200
_HALF = _S // 2
_NC = 2
_NS = 16
_NW = _NC * _NS
_ROWS_PER_W = _B // _NW
_NBUF = 4


def _body(ids_hbm, tok_hbm, pos_hbm, out_hbm, pos_sh, *scratch):
    idx = scratch[0:_NBUF]
    rows = scratch[_NBUF : 2 * _NBUF]
    sem_i = scratch[2 * _NBUF : 3 * _NBUF]
    sem_p = scratch[3 * _NBUF : 4 * _NBUF]
    sem_o = scratch[4 * _NBUF : 5 * _NBUF]
    sem_g = scratch[5 * _NBUF]

    sid = lax.axis_index("s")
    w = sid * _NC + lax.axis_index("c")
    base = w * _ROWS_PER_W

    @pl.when(sid == 0)
    def _():
        pltpu.sync_copy(pos_hbm, pos_sh)

    plsc.subcore_barrier()

    def start_idx(i, b):
        pltpu.async_copy(ids_hbm.at[base + i], idx[b], sem_i[b])

    def wait_idx(b):
        pltpu.make_async_copy(ids_hbm.at[0], idx[b], sem_i[b]).wait()

    def start_posinit(b):
        pltpu.async_copy(pos_sh, rows[b], sem_p[b])

    def wait_posinit(b):
        pltpu.make_async_copy(pos_sh, rows[b], sem_p[b]).wait()

    def start_out(i, b):
        pltpu.async_copy(rows[b], out_hbm.at[base + i], sem_o[b])

    def wait_out(b):
        pltpu.make_async_copy(rows[b], out_hbm.at[0], sem_o[b]).wait()

    for b in range(2):
        start_idx(b, b)
        start_posinit(b)

    @pl.loop(0, _ROWS_PER_W, step=_NBUF)
    def _(g):
        for b in range(_NBUF):
            i = g + b
            bn2 = (b + 2) % _NBUF
            wait_idx(b)
            wait_posinit(b)
            d0 = pltpu.async_copy(
                tok_hbm.at[idx[b].at[0]], rows[b].at[pl.ds(0, _HALF)], sem_g,
                add=True,
            )
            d1 = pltpu.async_copy(
                tok_hbm.at[idx[b].at[1]], rows[b].at[pl.ds(_HALF, _HALF)],
                sem_g, add=True,
            )

            @pl.when(i < _ROWS_PER_W - 2)
            def _():
                start_idx(i + 2, bn2)

            @pl.when(jnp.logical_and(i >= 2, i < _ROWS_PER_W - 2))
            def _():
                wait_out(bn2)

            @pl.when(i < _ROWS_PER_W - 2)
            def _():
                start_posinit(bn2)

            d0.wait()
            d1.wait()
            start_out(i, b)

    for b in range(_NBUF):
        wait_out(b)


@jax.jit
def _run(ids3, token_embedding, pos_s):
    mesh = plsc.VectorSubcoreMesh(
        core_axis_name="c", subcore_axis_name="s", num_cores=_NC, num_subcores=_NS
    )
    scratch = (
        [pltpu.VMEM_SHARED((_S, _HID), jnp.float32)]
        + [pltpu.VMEM((2, _HALF), jnp.int32) for _ in range(_NBUF)]
        + [pltpu.VMEM((_S, _HID), jnp.float32) for _ in range(_NBUF)]
        + [pltpu.SemaphoreType.DMA for _ in range(3 * _NBUF + 1)]
    )
    return pl.kernel(
        _body,
        out_type=jax.ShapeDtypeStruct((_B, _S, _HID), jnp.float32),
        mesh=mesh,
        compiler_params=pltpu.CompilerParams(use_tc_tiling_on_sc=False),
        scratch_types=scratch,
    )(ids3, token_embedding, pos_s)


def kernel(input_ids, token_embedding, position_embedding):
    ids3 = input_ids.reshape(_B, 2, _HALF)
    pos_s = position_embedding[:_S]
    return _run(ids3, token_embedding, pos_s)


# speedup vs baseline: 3.9288x; 1.0058x over previous
import functools

import jax
import jax.numpy as jnp
from jax import lax
from jax.experimental import pallas as pl
from jax.experimental.pallas import tpu as pltpu
from jax.experimental.pallas import tpu_sc as plsc
from jax.experimental.layout import Format, Layout

_HID = 64
_B = 4096
_S = 200
_HALF = _S // 2
_NC = 2
_NS = 16
_NW = _NC * _NS
_ROWS_PER_W = _B // _NW
_NBUF = 4


def _body(ids_hbm, tok_hbm, pos_hbm, out_hbm, pos_sh, *scratch):
    idx = scratch[0:_NBUF]
    rows = scratch[_NBUF : 2 * _NBUF]
    sem_i = scratch[2 * _NBUF : 3 * _NBUF]
    sem_p = scratch[3 * _NBUF : 4 * _NBUF]
    sem_o = scratch[4 * _NBUF : 5 * _NBUF]
    sem_g = scratch[5 * _NBUF]

    sid = lax.axis_index("s")
    w = sid * _NC + lax.axis_index("c")
    base = w * _ROWS_PER_W

    @pl.when(sid == 0)
    def _():
        pltpu.sync_copy(pos_hbm, pos_sh)

    plsc.subcore_barrier()

    def start_idx(i, b):
        pltpu.async_copy(ids_hbm.at[base + i], idx[b], sem_i[b])

    def wait_idx(b):
        pltpu.make_async_copy(ids_hbm.at[0], idx[b], sem_i[b]).wait()

    def start_posinit(b):
        pltpu.async_copy(pos_sh, rows[b], sem_p[b])

    def wait_posinit(b):
        pltpu.make_async_copy(pos_sh, rows[b], sem_p[b]).wait()

    def start_out(i, b):
        pltpu.async_copy(rows[b], out_hbm.at[base + i], sem_o[b])

    def wait_out(b):
        pltpu.make_async_copy(rows[b], out_hbm.at[0], sem_o[b]).wait()

    for b in range(2):
        start_idx(b, b)
        start_posinit(b)

    @pl.loop(0, _ROWS_PER_W, step=_NBUF)
    def _(g):
        for b in range(_NBUF):
            i = g + b
            bn2 = (b + 2) % _NBUF
            wait_idx(b)
            wait_posinit(b)
            d0 = pltpu.async_copy(
                tok_hbm.at[idx[b].at[0]], rows[b].at[pl.ds(0, _HALF)], sem_g,
                add=True,
            )
            d1 = pltpu.async_copy(
                tok_hbm.at[idx[b].at[1]], rows[b].at[pl.ds(_HALF, _HALF)],
                sem_g, add=True,
            )

            @pl.when(i < _ROWS_PER_W - 2)
            def _():
                start_idx(i + 2, bn2)

            @pl.when(jnp.logical_and(i >= 2, i < _ROWS_PER_W - 2))
            def _():
                wait_out(bn2)

            @pl.when(i < _ROWS_PER_W - 2)
            def _():
                start_posinit(bn2)

            d0.wait()
            d1.wait()
            start_out(i, b)

    for b in range(_NBUF):
        wait_out(b)


def _run_impl(ids3, token_embedding, pos_s):
    mesh = plsc.VectorSubcoreMesh(
        core_axis_name="c", subcore_axis_name="s", num_cores=_NC, num_subcores=_NS
    )
    scratch = (
        [pltpu.VMEM_SHARED((_S, _HID), jnp.float32)]
        + [pltpu.VMEM((2, _HALF), jnp.int32) for _ in range(_NBUF)]
        + [pltpu.VMEM((_S, _HID), jnp.float32) for _ in range(_NBUF)]
        + [pltpu.SemaphoreType.DMA for _ in range(3 * _NBUF + 1)]
    )
    return pl.kernel(
        _body,
        out_type=jax.ShapeDtypeStruct((_B, _S, _HID), jnp.float32),
        mesh=mesh,
        compiler_params=pltpu.CompilerParams(use_tc_tiling_on_sc=False),
        scratch_types=scratch,
    )(ids3, token_embedding, pos_s)


@functools.lru_cache(maxsize=1)
def _jitted_run():
    try:
        dev = jax.devices("tpu")[0]
        out_fmt = Format(
            Layout(major_to_minor=(0, 1, 2), tiling=()),
            jax.sharding.SingleDeviceSharding(dev),
        )
        return jax.jit(_run_impl, out_shardings=out_fmt)
    except RuntimeError:
        return jax.jit(_run_impl)


def kernel(input_ids, token_embedding, position_embedding):
    ids3 = input_ids.reshape(_B, 2, _HALF)
    pos_s = position_embedding[:_S]
    return _jitted_run()(ids3, token_embedding, pos_s)


# speedup vs baseline: 7.0582x; 1.7965x over previous
import functools

import jax
import jax.numpy as jnp
from jax import lax
from jax.experimental import pallas as pl
from jax.experimental.pallas import tpu as pltpu
from jax.experimental.pallas import tpu_sc as plsc

_HID = 64
_B = 4096
_S = 200
_HALF = _S // 2
_NC = 2
_NS = 16
_NW = _NC * _NS
_ROWS_PER_W = _B // _NW
_NBUF = 4


def _body(ids_hbm, tok_hbm, pos_hbm, out_hbm, pos_sh, *scratch):
    idx = scratch[0:_NBUF]
    rows = scratch[_NBUF : 2 * _NBUF]
    sem_i = scratch[2 * _NBUF : 3 * _NBUF]
    sem_p = scratch[3 * _NBUF : 4 * _NBUF]
    sem_o = scratch[4 * _NBUF : 5 * _NBUF]
    sem_g = scratch[5 * _NBUF]

    sid = lax.axis_index("s")
    w = sid * _NC + lax.axis_index("c")
    base = w * _ROWS_PER_W

    @pl.when(sid == 0)
    def _():
        pltpu.sync_copy(pos_hbm, pos_sh)

    plsc.subcore_barrier()

    def start_idx(i, b):
        pltpu.async_copy(ids_hbm.at[base + i], idx[b], sem_i[b])

    def wait_idx(b):
        pltpu.make_async_copy(ids_hbm.at[0], idx[b], sem_i[b]).wait()

    def start_posinit(b):
        pltpu.async_copy(pos_sh, rows[b], sem_p[b])

    def wait_posinit(b):
        pltpu.make_async_copy(pos_sh, rows[b], sem_p[b]).wait()

    def start_out(i, b):
        pltpu.async_copy(rows[b], out_hbm.at[base + i], sem_o[b])

    def wait_out(b):
        pltpu.make_async_copy(rows[b], out_hbm.at[0], sem_o[b]).wait()

    for b in range(2):
        start_idx(b, b)
        start_posinit(b)

    @pl.loop(0, _ROWS_PER_W, step=_NBUF)
    def _(g):
        for b in range(_NBUF):
            i = g + b
            bn2 = (b + 2) % _NBUF
            wait_idx(b)
            wait_posinit(b)
            d0 = pltpu.async_copy(
                tok_hbm.at[idx[b].at[0]], rows[b].at[pl.ds(0, _HALF)], sem_g,
                add=True,
            )
            d1 = pltpu.async_copy(
                tok_hbm.at[idx[b].at[1]], rows[b].at[pl.ds(_HALF, _HALF)],
                sem_g, add=True,
            )

            @pl.when(i < _ROWS_PER_W - 2)
            def _():
                start_idx(i + 2, bn2)

            @pl.when(jnp.logical_and(i >= 2, i < _ROWS_PER_W - 2))
            def _():
                wait_out(bn2)

            @pl.when(i < _ROWS_PER_W - 2)
            def _():
                start_posinit(bn2)

            d0.wait()
            d1.wait()
            start_out(i, b)

    for b in range(_NBUF):
        wait_out(b)


def _sc_gather(ids3, token_embedding, pos_s):
    mesh = plsc.VectorSubcoreMesh(
        core_axis_name="c", subcore_axis_name="s", num_cores=_NC, num_subcores=_NS
    )
    scratch = (
        [pltpu.VMEM_SHARED((_S, _HID), jnp.float32)]
        + [pltpu.VMEM((2, _HALF), jnp.int32) for _ in range(_NBUF)]
        + [pltpu.VMEM((_S, _HID), jnp.float32) for _ in range(_NBUF)]
        + [pltpu.SemaphoreType.DMA for _ in range(3 * _NBUF + 1)]
    )
    return pl.kernel(
        _body,
        out_type=jax.ShapeDtypeStruct((_B, _S, _HID), jnp.float32),
        mesh=mesh,
        compiler_params=pltpu.CompilerParams(use_tc_tiling_on_sc=False),
        scratch_types=scratch,
    )(ids3, token_embedding, pos_s)


_BB = 128


def _tc_transpose_body(x_ref, o_ref):
    x = x_ref[...]
    o_ref[...] = x.reshape(_BB, _S * _HID // 128 * 128).T.reshape(_S, _HID, _BB)


def _tc_transpose(t2):
    grid = _B // _BB
    return pl.pallas_call(
        _tc_transpose_body,
        out_shape=jax.ShapeDtypeStruct((_S, _HID, _B), jnp.float32),
        grid=(grid,),
        in_specs=[pl.BlockSpec((_BB * 100, 128), lambda g: (g, 0))],
        out_specs=pl.BlockSpec((_S, _HID, _BB), lambda g: (0, 0, g)),
    )(t2)


def _run_impl(ids3, token_embedding, pos_s):
    lin = _sc_gather(ids3, token_embedding, pos_s)
    t2 = lin.reshape(_B * _S * _HID // 128, 128)
    o_t = _tc_transpose(t2)
    return jnp.transpose(o_t, (2, 0, 1))


def kernel(input_ids, token_embedding, position_embedding):
    ids3 = input_ids.reshape(_B, 2, _HALF)
    pos_s = position_embedding[:_S]
    return _run_impl(ids3, token_embedding, pos_s)


# speedup vs baseline: 7.3112x; 1.0359x over previous
import functools

import jax
import jax.numpy as jnp
from jax import lax
from jax.experimental import pallas as pl
from jax.experimental.pallas import tpu as pltpu
from jax.experimental.pallas import tpu_sc as plsc

_HID = 64
_B = 4096
_S = 200
_HALF = _S // 2
_NC = 2
_NS = 16
_NW = _NC * _NS
_K = 4
_BCH = _B // _K
_ROWS_PER_W = _BCH // _NW
_NBUF = 4


def _body(chunk_base, ids_hbm, tok_hbm, pos_hbm, out_hbm, pos_sh, *scratch):
    idx = scratch[0:_NBUF]
    rows = scratch[_NBUF : 2 * _NBUF]
    sem_i = scratch[2 * _NBUF : 3 * _NBUF]
    sem_p = scratch[3 * _NBUF : 4 * _NBUF]
    sem_o = scratch[4 * _NBUF : 5 * _NBUF]
    sem_g = scratch[5 * _NBUF]

    sid = lax.axis_index("s")
    w = sid * _NC + lax.axis_index("c")
    base = w * _ROWS_PER_W

    @pl.when(sid == 0)
    def _():
        pltpu.sync_copy(pos_hbm, pos_sh)

    plsc.subcore_barrier()

    def start_idx(i, b):
        pltpu.async_copy(ids_hbm.at[chunk_base + base + i], idx[b], sem_i[b])

    def wait_idx(b):
        pltpu.make_async_copy(ids_hbm.at[0], idx[b], sem_i[b]).wait()

    def start_posinit(b):
        pltpu.async_copy(pos_sh, rows[b], sem_p[b])

    def wait_posinit(b):
        pltpu.make_async_copy(pos_sh, rows[b], sem_p[b]).wait()

    def start_out(i, b):
        pltpu.async_copy(rows[b], out_hbm.at[base + i], sem_o[b])

    def wait_out(b):
        pltpu.make_async_copy(rows[b], out_hbm.at[0], sem_o[b]).wait()

    for b in range(2):
        start_idx(b, b)
        start_posinit(b)

    @pl.loop(0, _ROWS_PER_W, step=_NBUF)
    def _(g):
        for b in range(_NBUF):
            i = g + b
            bn2 = (b + 2) % _NBUF
            wait_idx(b)
            wait_posinit(b)
            d0 = pltpu.async_copy(
                tok_hbm.at[idx[b].at[0]], rows[b].at[pl.ds(0, _HALF)], sem_g,
                add=True,
            )
            d1 = pltpu.async_copy(
                tok_hbm.at[idx[b].at[1]], rows[b].at[pl.ds(_HALF, _HALF)],
                sem_g, add=True,
            )

            @pl.when(i < _ROWS_PER_W - 2)
            def _():
                start_idx(i + 2, bn2)

            @pl.when(jnp.logical_and(i >= 2, i < _ROWS_PER_W - 2))
            def _():
                wait_out(bn2)

            @pl.when(i < _ROWS_PER_W - 2)
            def _():
                start_posinit(bn2)

            d0.wait()
            d1.wait()
            start_out(i, b)

    for b in range(_NBUF):
        wait_out(b)


def _sc_gather(k, ids3, token_embedding, pos_s):
    mesh = plsc.VectorSubcoreMesh(
        core_axis_name="c", subcore_axis_name="s", num_cores=_NC, num_subcores=_NS
    )
    scratch = (
        [pltpu.VMEM_SHARED((_S, _HID), jnp.float32)]
        + [pltpu.VMEM((2, _HALF), jnp.int32) for _ in range(_NBUF)]
        + [pltpu.VMEM((_S, _HID), jnp.float32) for _ in range(_NBUF)]
        + [pltpu.SemaphoreType.DMA for _ in range(3 * _NBUF + 1)]
    )
    return pl.kernel(
        functools.partial(_body, k * _BCH),
        out_type=jax.ShapeDtypeStruct((_BCH, _S, _HID), jnp.float32),
        mesh=mesh,
        compiler_params=pltpu.CompilerParams(use_tc_tiling_on_sc=False),
        scratch_types=scratch,
        name=f"sc_gather_{k}",
    )(ids3, token_embedding, pos_s)


_BB = 128
_GRID = _BCH // _BB


def _tc_transpose_first_body(x_ref, o_ref):
    x = x_ref[...]
    o_ref[...] = x.reshape(_BB, _S * _HID).T.reshape(_S, _HID, _BB)


def _tc_transpose_update_body(acc_ref, x_ref, o_ref):
    del acc_ref
    x = x_ref[...]
    o_ref[...] = x.reshape(_BB, _S * _HID).T.reshape(_S, _HID, _BB)


def _tc_transpose_chunk(k, acc, t2):
    out_shape = jax.ShapeDtypeStruct((_S, _HID, _B), jnp.float32)
    x_spec = pl.BlockSpec((_BB * 100, 128), lambda g: (g, 0))
    o_spec = pl.BlockSpec((_S, _HID, _BB), lambda g, _k=k: (0, 0, _k * _GRID + g))
    if acc is None:
        return pl.pallas_call(
            _tc_transpose_first_body,
            out_shape=out_shape,
            grid=(_GRID,),
            in_specs=[x_spec],
            out_specs=o_spec,
            name="tc_transpose_0",
        )(t2)
    return pl.pallas_call(
        _tc_transpose_update_body,
        out_shape=out_shape,
        grid=(_GRID,),
        in_specs=[pl.BlockSpec(memory_space=pl.ANY), x_spec],
        out_specs=o_spec,
        input_output_aliases={0: 0},
        name=f"tc_transpose_{k}",
    )(acc, t2)


def _run_impl(ids3, token_embedding, pos_s):
    acc = None
    for k in range(_K):
        lin = _sc_gather(k, ids3, token_embedding, pos_s)
        t2 = lin.reshape(_BCH * _S * _HID // 128, 128)
        acc = _tc_transpose_chunk(k, acc, t2)
    return jnp.transpose(acc, (2, 0, 1))


def kernel(input_ids, token_embedding, position_embedding):
    ids3 = input_ids.reshape(_B, 2, _HALF)
    pos_s = position_embedding[:_S]
    return _run_impl(ids3, token_embedding, pos_s)


# speedup vs baseline: 7.5826x; 1.0371x over previous
import functools

import jax
import jax.numpy as jnp
from jax import lax
from jax.experimental import pallas as pl
from jax.experimental.pallas import tpu as pltpu
from jax.experimental.pallas import tpu_sc as plsc

_HID = 64
_B = 4096
_S = 200
_C0 = 104
_C1 = _S - _C0
_NC = 2
_NS = 16
_NW = _NC * _NS
_K = 4
_BCH = _B // _K
_ROWS_PER_W = _BCH // _NW
_NBUF = 4


def _body(chunk_base, ids_hbm, tok_hbm, pos_hbm, out_hbm, pos_sh, *scratch):
    idx = scratch[0:_NBUF]
    rows = scratch[_NBUF : 2 * _NBUF]
    sem_i = scratch[2 * _NBUF : 3 * _NBUF]
    sem_p = scratch[3 * _NBUF : 4 * _NBUF]
    sem_o = scratch[4 * _NBUF : 5 * _NBUF]
    sem_g = scratch[5 * _NBUF]

    sid = lax.axis_index("s")
    w = sid * _NC + lax.axis_index("c")
    base = w * _ROWS_PER_W

    @pl.when(sid == 0)
    def _():
        pltpu.sync_copy(pos_hbm, pos_sh)

    plsc.subcore_barrier()

    def start_idx(i, b):
        pltpu.async_copy(ids_hbm.at[chunk_base + base + i], idx[b], sem_i[b])

    def wait_idx(b):
        pltpu.make_async_copy(ids_hbm.at[0], idx[b], sem_i[b]).wait()

    def start_posinit(b):
        pltpu.async_copy(pos_sh, rows[b], sem_p[b])

    def wait_posinit(b):
        pltpu.make_async_copy(pos_sh, rows[b], sem_p[b]).wait()

    def start_out(i, b):
        pltpu.async_copy(rows[b], out_hbm.at[base + i], sem_o[b])

    def wait_out(b):
        pltpu.make_async_copy(rows[b], out_hbm.at[0], sem_o[b]).wait()

    for b in range(2):
        start_idx(b, b)
        start_posinit(b)

    @pl.loop(0, _ROWS_PER_W, step=_NBUF)
    def _(g):
        for b in range(_NBUF):
            i = g + b
            bn2 = (b + 2) % _NBUF
            wait_idx(b)
            wait_posinit(b)
            d0 = pltpu.async_copy(
                tok_hbm.at[idx[b].at[pl.ds(0, _C0)]],
                rows[b].at[pl.ds(0, _C0)], sem_g, add=True,
            )
            d1 = pltpu.async_copy(
                tok_hbm.at[idx[b].at[pl.ds(_C0, _C1)]],
                rows[b].at[pl.ds(_C0, _C1)], sem_g, add=True,
            )

            @pl.when(i < _ROWS_PER_W - 2)
            def _():
                start_idx(i + 2, bn2)

            @pl.when(jnp.logical_and(i >= 2, i < _ROWS_PER_W - 2))
            def _():
                wait_out(bn2)

            @pl.when(i < _ROWS_PER_W - 2)
            def _():
                start_posinit(bn2)

            d0.wait()
            d1.wait()
            start_out(i, b)

    for b in range(_NBUF):
        wait_out(b)


def _sc_gather(k, ids3, token_embedding, pos_s):
    mesh = plsc.VectorSubcoreMesh(
        core_axis_name="c", subcore_axis_name="s", num_cores=_NC, num_subcores=_NS
    )
    scratch = (
        [pltpu.VMEM_SHARED((_S, _HID), jnp.float32)]
        + [pltpu.VMEM((_S,), jnp.int32) for _ in range(_NBUF)]
        + [pltpu.VMEM((_S, _HID), jnp.float32) for _ in range(_NBUF)]
        + [pltpu.SemaphoreType.DMA for _ in range(3 * _NBUF + 1)]
    )
    return pl.kernel(
        functools.partial(_body, k * _BCH),
        out_type=jax.ShapeDtypeStruct((_BCH, _S, _HID), jnp.float32),
        mesh=mesh,
        compiler_params=pltpu.CompilerParams(use_tc_tiling_on_sc=False),
        scratch_types=scratch,
        name=f"sc_gather_{k}",
    )(ids3, token_embedding, pos_s)


_BB = 128
_GRID = _BCH // _BB


def _tc_transpose_first_body(x_ref, o_ref):
    x = x_ref[...]
    o_ref[...] = x.reshape(_BB, _S * _HID).T.reshape(_S, _HID, _BB)


def _tc_transpose_update_body(acc_ref, x_ref, o_ref):
    del acc_ref
    x = x_ref[...]
    o_ref[...] = x.reshape(_BB, _S * _HID).T.reshape(_S, _HID, _BB)


def _tc_transpose_chunk(k, acc, t2):
    out_shape = jax.ShapeDtypeStruct((_S, _HID, _B), jnp.float32)
    x_spec = pl.BlockSpec((_BB * 100, 128), lambda g: (g, 0))
    o_spec = pl.BlockSpec((_S, _HID, _BB), lambda g, _k=k: (0, 0, _k * _GRID + g))
    if acc is None:
        return pl.pallas_call(
            _tc_transpose_first_body,
            out_shape=out_shape,
            grid=(_GRID,),
            in_specs=[x_spec],
            out_specs=o_spec,
            name="tc_transpose_0",
        )(t2)
    return pl.pallas_call(
        _tc_transpose_update_body,
        out_shape=out_shape,
        grid=(_GRID,),
        in_specs=[pl.BlockSpec(memory_space=pl.ANY), x_spec],
        out_specs=o_spec,
        input_output_aliases={0: 0},
        name=f"tc_transpose_{k}",
    )(acc, t2)


def _run_impl(input_ids, token_embedding, pos_s):
    tok_lin = token_embedding.reshape(50000, 128).reshape(100000, _HID)
    acc = None
    for k in range(_K):
        lin = _sc_gather(k, input_ids, tok_lin, pos_s)
        t2 = lin.reshape(_BCH * _S * _HID // 128, 128)
        acc = _tc_transpose_chunk(k, acc, t2)
    return jnp.transpose(acc, (2, 0, 1))


def kernel(input_ids, token_embedding, position_embedding):
    pos_s = position_embedding[:_S]
    return _run_impl(input_ids, token_embedding, pos_s)


# speedup vs baseline: 7.6809x; 1.0130x over previous
import functools

import jax
import jax.numpy as jnp
from jax import lax
from jax.experimental import pallas as pl
from jax.experimental.pallas import tpu as pltpu
from jax.experimental.pallas import tpu_sc as plsc

_HID = 64
_B = 4096
_S = 200
_C0 = 104
_C1 = _S - _C0
_NC = 2
_NS = 16
_NW = _NC * _NS
_K = 4
_BCH = _B // _K
_ROWS_PER_W = _BCH // _NW
_NBUF = 4


def _body(chunk_base, ids_hbm, tok_hbm, pos_hbm, out_hbm, pos_sh, *scratch):
    idx = scratch[0:_NBUF]
    rows = scratch[_NBUF : 2 * _NBUF]
    sem_i = scratch[2 * _NBUF : 3 * _NBUF]
    sem_p = scratch[3 * _NBUF : 4 * _NBUF]
    sem_o = scratch[4 * _NBUF : 5 * _NBUF]
    sem_g = scratch[5 * _NBUF]

    sid = lax.axis_index("s")
    w = sid * _NC + lax.axis_index("c")
    base = w * _ROWS_PER_W

    @pl.when(sid == 0)
    def _():
        pltpu.sync_copy(pos_hbm, pos_sh)

    plsc.subcore_barrier()

    def start_idx(i, b):
        pltpu.async_copy(ids_hbm.at[chunk_base + base + i], idx[b], sem_i[b])

    def wait_idx(b):
        pltpu.make_async_copy(ids_hbm.at[0], idx[b], sem_i[b]).wait()

    def start_posinit(b):
        pltpu.async_copy(pos_sh, rows[b], sem_p[b])

    def wait_posinit(b):
        pltpu.make_async_copy(pos_sh, rows[b], sem_p[b]).wait()

    def start_out(i, b):
        pltpu.async_copy(rows[b], out_hbm.at[base + i], sem_o[b])

    def wait_out(b):
        pltpu.make_async_copy(rows[b], out_hbm.at[0], sem_o[b]).wait()

    for b in range(2):
        start_idx(b, b)
        start_posinit(b)

    @pl.loop(0, _ROWS_PER_W, step=_NBUF)
    def _(g):
        for b in range(_NBUF):
            i = g + b
            bn2 = (b + 2) % _NBUF
            wait_idx(b)
            wait_posinit(b)
            d0 = pltpu.async_copy(
                tok_hbm.at[idx[b].at[pl.ds(0, _C0)]],
                rows[b].at[pl.ds(0, _C0)], sem_g, add=True,
            )
            d1 = pltpu.async_copy(
                tok_hbm.at[idx[b].at[pl.ds(_C0, _C1)]],
                rows[b].at[pl.ds(_C0, _C1)], sem_g, add=True,
            )

            @pl.when(i < _ROWS_PER_W - 2)
            def _():
                start_idx(i + 2, bn2)

            @pl.when(jnp.logical_and(i >= 2, i < _ROWS_PER_W - 2))
            def _():
                wait_out(bn2)

            @pl.when(i < _ROWS_PER_W - 2)
            def _():
                start_posinit(bn2)

            d0.wait()
            d1.wait()
            start_out(i, b)

    for b in range(_NBUF):
        wait_out(b)


def _sc_gather(k, ids3, token_embedding, pos_s):
    mesh = plsc.VectorSubcoreMesh(
        core_axis_name="c", subcore_axis_name="s", num_cores=_NC, num_subcores=_NS
    )
    scratch = (
        [pltpu.VMEM_SHARED((_S, _HID), jnp.float32)]
        + [pltpu.VMEM((_S,), jnp.int32) for _ in range(_NBUF)]
        + [pltpu.VMEM((_S, _HID), jnp.float32) for _ in range(_NBUF)]
        + [pltpu.SemaphoreType.DMA for _ in range(3 * _NBUF + 1)]
    )
    return pl.kernel(
        functools.partial(_body, k * _BCH),
        out_type=jax.ShapeDtypeStruct((_BCH, _S, _HID), jnp.float32),
        mesh=mesh,
        compiler_params=pltpu.CompilerParams(use_tc_tiling_on_sc=False),
        scratch_types=scratch,
        name=f"sc_gather_{k}",
    )(ids3, token_embedding, pos_s)


_BB = 128
_GRID = _BCH // _BB


def _tc_transpose_first_body(x_ref, o_ref):
    x = x_ref[...]
    o_ref[...] = x.reshape(_BB, _S * _HID).T.reshape(_S, _HID, _BB)


def _tc_transpose_update_body(acc_ref, x_ref, o_ref):
    del acc_ref
    x = x_ref[...]
    o_ref[...] = x.reshape(_BB, _S * _HID).T.reshape(_S, _HID, _BB)


def _tc_transpose_chunk(k, acc, t2):
    out_shape = jax.ShapeDtypeStruct((_S, _HID, _B), jnp.float32)
    x_spec = pl.BlockSpec((_BB * 100, 128), lambda g: (g, 0))
    o_spec = pl.BlockSpec((_S, _HID, _BB), lambda g, _k=k: (0, 0, _k * _GRID + g))
    if acc is None:
        return pl.pallas_call(
            _tc_transpose_first_body,
            out_shape=out_shape,
            grid=(_GRID,),
            in_specs=[x_spec],
            out_specs=o_spec,
            name="tc_transpose_0",
        )(t2)
    return pl.pallas_call(
        _tc_transpose_update_body,
        out_shape=out_shape,
        grid=(_GRID,),
        in_specs=[pl.BlockSpec(memory_space=pl.ANY), x_spec],
        out_specs=o_spec,
        input_output_aliases={0: 0},
        name=f"tc_transpose_{k}",
    )(acc, t2)


_VPAD = 102400
_VH = _VPAD // 2
_W = _VH // 8


def _tc_lin_table_body(xl_ref, xr_ref, o_ref):
    o_ref[:, 0:_HID] = xl_ref[...].T
    o_ref[:, _HID:128] = xr_ref[...].T


def _tc_lin_table(tok_t):
    return pl.pallas_call(
        _tc_lin_table_body,
        out_shape=jax.ShapeDtypeStruct((_VH, 128), jnp.float32),
        grid=(8,),
        in_specs=[
            pl.BlockSpec((_HID, _W), lambda g: (0, g)),
            pl.BlockSpec((_HID, _W), lambda g: (0, g + 8)),
        ],
        out_specs=pl.BlockSpec((_W, 128), lambda g: (g, 0)),
        name="tc_lin_table",
    )(tok_t, tok_t)


def _run_impl(input_ids, token_embedding, pos_s):
    tok_t = jnp.pad(jnp.transpose(token_embedding), ((0, 0), (0, _VPAD - 100000)))
    tok_lin = _tc_lin_table(tok_t).reshape(_VPAD, _HID)
    ids_m = jnp.where(
        input_ids < _VH, input_ids * 2, (input_ids - _VH) * 2 + 1
    ).astype(jnp.int32)
    acc = None
    for k in range(_K):
        lin = _sc_gather(k, ids_m, tok_lin, pos_s)
        t2 = lin.reshape(_BCH * _S * _HID // 128, 128)
        acc = _tc_transpose_chunk(k, acc, t2)
    return jnp.transpose(acc, (2, 0, 1))


def kernel(input_ids, token_embedding, position_embedding):
    pos_s = position_embedding[:_S]
    return _run_impl(input_ids, token_embedding, pos_s)


# speedup vs baseline: 7.7353x; 1.0071x over previous
import functools

import jax
import jax.numpy as jnp
from jax import lax
from jax.experimental import pallas as pl
from jax.experimental.pallas import tpu as pltpu
from jax.experimental.pallas import tpu_sc as plsc

_HID = 64
_B = 4096
_S = 200
_C0 = 104
_C1 = _S - _C0
_NC = 2
_NS = 16
_NW = _NC * _NS
_K = 4
_CHUNKS = (768, 1280, 1280, 768)
_STARTS = (0, 768, 2048, 3328)
_NBUF = 4


def _body(chunk_base, rows_per_w, ids_hbm, tok_hbm, pos_hbm, out_hbm, pos_sh, *scratch):
    idx = scratch[0:_NBUF]
    rows = scratch[_NBUF : 2 * _NBUF]
    sem_i = scratch[2 * _NBUF : 3 * _NBUF]
    sem_p = scratch[3 * _NBUF : 4 * _NBUF]
    sem_o = scratch[4 * _NBUF : 5 * _NBUF]
    sem_g = scratch[5 * _NBUF]

    sid = lax.axis_index("s")
    w = sid * _NC + lax.axis_index("c")
    base = w * rows_per_w

    @pl.when(sid == 0)
    def _():
        pltpu.sync_copy(pos_hbm, pos_sh)

    plsc.subcore_barrier()

    def start_idx(i, b):
        pltpu.async_copy(ids_hbm.at[chunk_base + base + i], idx[b], sem_i[b])

    def wait_idx(b):
        pltpu.make_async_copy(ids_hbm.at[0], idx[b], sem_i[b]).wait()

    def start_posinit(b):
        pltpu.async_copy(pos_sh, rows[b], sem_p[b])

    def wait_posinit(b):
        pltpu.make_async_copy(pos_sh, rows[b], sem_p[b]).wait()

    def start_out(i, b):
        pltpu.async_copy(rows[b], out_hbm.at[base + i], sem_o[b])

    def wait_out(b):
        pltpu.make_async_copy(rows[b], out_hbm.at[0], sem_o[b]).wait()

    for b in range(2):
        start_idx(b, b)
        start_posinit(b)

    @pl.loop(0, rows_per_w, step=_NBUF)
    def _(g):
        for b in range(_NBUF):
            i = g + b
            bn2 = (b + 2) % _NBUF
            wait_idx(b)
            wait_posinit(b)
            d0 = pltpu.async_copy(
                tok_hbm.at[idx[b].at[pl.ds(0, _C0)]],
                rows[b].at[pl.ds(0, _C0)], sem_g, add=True,
            )
            d1 = pltpu.async_copy(
                tok_hbm.at[idx[b].at[pl.ds(_C0, _C1)]],
                rows[b].at[pl.ds(_C0, _C1)], sem_g, add=True,
            )

            @pl.when(i < rows_per_w - 2)
            def _():
                start_idx(i + 2, bn2)

            @pl.when(jnp.logical_and(i >= 2, i < rows_per_w - 2))
            def _():
                wait_out(bn2)

            @pl.when(i < rows_per_w - 2)
            def _():
                start_posinit(bn2)

            d0.wait()
            d1.wait()
            start_out(i, b)

    for b in range(_NBUF):
        wait_out(b)


def _sc_gather(k, ids3, token_embedding, pos_s):
    mesh = plsc.VectorSubcoreMesh(
        core_axis_name="c", subcore_axis_name="s", num_cores=_NC, num_subcores=_NS
    )
    scratch = (
        [pltpu.VMEM_SHARED((_S, _HID), jnp.float32)]
        + [pltpu.VMEM((_S,), jnp.int32) for _ in range(_NBUF)]
        + [pltpu.VMEM((_S, _HID), jnp.float32) for _ in range(_NBUF)]
        + [pltpu.SemaphoreType.DMA for _ in range(3 * _NBUF + 1)]
    )
    bch = _CHUNKS[k]
    return pl.kernel(
        functools.partial(_body, _STARTS[k], bch // _NW),
        out_type=jax.ShapeDtypeStruct((bch, _S, _HID), jnp.float32),
        mesh=mesh,
        compiler_params=pltpu.CompilerParams(use_tc_tiling_on_sc=False),
        scratch_types=scratch,
        name=f"sc_gather_{k}",
    )(ids3, token_embedding, pos_s)


_BB = 128


def _tc_transpose_first_body(x_ref, o_ref):
    x = x_ref[...]
    o_ref[...] = x.reshape(_BB, _S * _HID).T.reshape(_S, _HID, _BB)


def _tc_transpose_update_body(acc_ref, x_ref, o_ref):
    del acc_ref
    x = x_ref[...]
    o_ref[...] = x.reshape(_BB, _S * _HID).T.reshape(_S, _HID, _BB)


def _tc_transpose_chunk(k, acc, t2):
    out_shape = jax.ShapeDtypeStruct((_S, _HID, _B), jnp.float32)
    grid = _CHUNKS[k] // _BB
    boff = _STARTS[k] // _BB
    x_spec = pl.BlockSpec((_BB * 100, 128), lambda g: (g, 0))
    o_spec = pl.BlockSpec((_S, _HID, _BB), lambda g, _o=boff: (0, 0, _o + g))
    if acc is None:
        return pl.pallas_call(
            _tc_transpose_first_body,
            out_shape=out_shape,
            grid=(grid,),
            in_specs=[x_spec],
            out_specs=o_spec,
            name="tc_transpose_0",
        )(t2)
    return pl.pallas_call(
        _tc_transpose_update_body,
        out_shape=out_shape,
        grid=(grid,),
        in_specs=[pl.BlockSpec(memory_space=pl.ANY), x_spec],
        out_specs=o_spec,
        input_output_aliases={0: 0},
        name=f"tc_transpose_{k}",
    )(acc, t2)


_VPAD = 102400
_VH = _VPAD // 2
_W = _VH // 8


def _tc_lin_table_body(xl_ref, xr_ref, o_ref):
    o_ref[:, 0:_HID] = xl_ref[...].T
    o_ref[:, _HID:128] = xr_ref[...].T


def _tc_lin_table(tok_t):
    return pl.pallas_call(
        _tc_lin_table_body,
        out_shape=jax.ShapeDtypeStruct((_VH, 128), jnp.float32),
        grid=(8,),
        in_specs=[
            pl.BlockSpec((_HID, _W), lambda g: (0, g)),
            pl.BlockSpec((_HID, _W), lambda g: (0, g + 8)),
        ],
        out_specs=pl.BlockSpec((_W, 128), lambda g: (g, 0)),
        name="tc_lin_table",
    )(tok_t, tok_t)


def _run_impl(input_ids, token_embedding, pos_s):
    tok_t = jnp.pad(jnp.transpose(token_embedding), ((0, 0), (0, _VPAD - 100000)))
    tok_lin = _tc_lin_table(tok_t).reshape(_VPAD, _HID)
    ids_m = jnp.where(
        input_ids < _VH, input_ids * 2, (input_ids - _VH) * 2 + 1
    ).astype(jnp.int32)
    acc = None
    for k in range(_K):
        lin = _sc_gather(k, ids_m, tok_lin, pos_s)
        t2 = lin.reshape(_CHUNKS[k] * _S * _HID // 128, 128)
        acc = _tc_transpose_chunk(k, acc, t2)
    return jnp.transpose(acc, (2, 0, 1))


def kernel(input_ids, token_embedding, position_embedding):
    pos_s = position_embedding[:_S]
    return _run_impl(input_ids, token_embedding, pos_s)
